# Initial kernel scaffold; baseline (speedup 1.0000x reference)
import jax
import jax.numpy as jnp
from jax.experimental import pallas as pl


def kernel(x_s, label, confounder_queue, sel_idx):
    raise NotImplementedError("write your pallas kernel here")


# speedup vs baseline: 4.4823x; 4.4823x over previous
import functools

import jax
import jax.numpy as jnp
from jax import lax
from jax.experimental import pallas as pl
from jax.experimental.pallas import tpu as pltpu
from jax.experimental.pallas import tpu_sc as plsc

NUM_CLASSES = 10
K = 2000
N = 16
ROW = 3 * 32 * 32
NC, NS, L = 2, 16, 16
NW = NC * NS
BS = 64
ROWS_PER_W = BS * (NUM_CLASSES - 1) * N // NW
CHUNKS = ROWS_PER_W // L
PER_SAMPLE_CHUNKS = (NUM_CLASSES - 1) * N // L


def _gather_body(table_hbm, labelrep_hbm, sel_hbm, out_hbm,
                 labelrep_v, selidx_v, sel16_v, idx_v, rows_v, sem):
    w = lax.axis_index("s") * NC + lax.axis_index("c")
    base = w * ROWS_PER_W
    pltpu.sync_copy(labelrep_hbm, labelrep_v)
    lanes = lax.iota(jnp.int32, L)

    @pl.loop(0, CHUNKS)
    def _chunk(c):
        j = (BS // NW) * w + c // PER_SAMPLE_CHUNKS
        q = c % PER_SAMPLE_CHUNKS
        lbl = labelrep_v[pl.ds(j * L, L)]
        ge = 1 - (lax.shift_right_logical(q - lbl, 31) & 1)
        cls = q + ge
        selidx_v[...] = j * (NUM_CLASSES * N) + cls * N + lanes
        pltpu.async_copy(sel_hbm.at[selidx_v], sel16_v, sem).wait()
        idx_v[...] = cls * K + sel16_v[...]
        pltpu.async_copy(table_hbm.at[idx_v], rows_v, sem).wait()
        pltpu.sync_copy(rows_v, out_hbm.at[pl.ds(base + c * L, L)])


def kernel(x_s, label, confounder_queue, sel_idx):
    bs = x_s.shape[0]
    table = confounder_queue.reshape(NUM_CLASSES * K, ROW)
    sel_flat = sel_idx.reshape(-1).astype(jnp.int32)
    label_rep = jnp.repeat(label.astype(jnp.int32), L)
    mesh = plsc.VectorSubcoreMesh(
        core_axis_name="c", subcore_axis_name="s",
        num_cores=NC, num_subcores=NS)
    run = pl.kernel(
        _gather_body,
        out_type=jax.ShapeDtypeStruct((bs * (NUM_CLASSES - 1) * N, ROW),
                                      jnp.float32),
        mesh=mesh,
        scratch_types=[
            pltpu.VMEM((BS * L,), jnp.int32),
            pltpu.VMEM((L,), jnp.int32),
            pltpu.VMEM((L,), jnp.int32),
            pltpu.VMEM((L,), jnp.int32),
            pltpu.VMEM((L, ROW), jnp.float32),
            pltpu.SemaphoreType.DMA,
        ],
    )
    out = run(table, label_rep, sel_flat)
    return out.reshape(bs, (NUM_CLASSES - 1) * N,
                       confounder_queue.shape[2], confounder_queue.shape[3],
                       confounder_queue.shape[4])


# speedup vs baseline: 4.5418x; 1.0133x over previous
import functools

import jax
import jax.numpy as jnp
from jax import lax
from jax.experimental import pallas as pl
from jax.experimental.pallas import tpu as pltpu
from jax.experimental.pallas import tpu_sc as plsc

NUM_CLASSES = 10
K = 2000
N = 16
ROW = 3 * 32 * 32
NC, NS, L = 2, 16, 16
NW = NC * NS
BS = 64
ROWS_PER_W = BS * (NUM_CLASSES - 1) * N // NW
CHUNKS = ROWS_PER_W // L
PER_SAMPLE_CHUNKS = (NUM_CLASSES - 1) * N // L
SEL_DMA = 96
SEL_DMAS = ROWS_PER_W // SEL_DMA


def _gather_body(table_hbm, labelrep_hbm, sel_hbm, out_hbm,
                 labelrep_v, selall_v, selval_v, clsk_v, flatidx_v,
                 rows0_v, rows1_v, gsem0, gsem1, ssem0, ssem1):
    w = lax.axis_index("s") * NC + lax.axis_index("c")
    base = w * ROWS_PER_W
    pltpu.sync_copy(labelrep_hbm, labelrep_v)
    lanes = lax.iota(jnp.int32, L)

    @pl.loop(0, CHUNKS)
    def _idx(c):
        j = (BS // NW) * w + c // PER_SAMPLE_CHUNKS
        q = c % PER_SAMPLE_CHUNKS
        lbl = labelrep_v[pl.ds(j * L, L)]
        ge = 1 - (lax.shift_right_logical(q - lbl, 31) & 1)
        cls = q + ge
        selall_v[pl.ds(c * L, L)] = j * (NUM_CLASSES * N) + cls * N + lanes
        clsk_v[pl.ds(c * L, L)] = cls * K

    descs = [
        pltpu.async_copy(
            sel_hbm.at[selall_v.at[pl.ds(g * SEL_DMA, SEL_DMA)]],
            selval_v.at[pl.ds(g * SEL_DMA, SEL_DMA)], gsem0)
        for g in range(SEL_DMAS)
    ]
    for d in descs:
        d.wait()

    @pl.loop(0, CHUNKS)
    def _flat(c):
        s = pl.ds(c * L, L)
        flatidx_v[s] = clsk_v[s] + selval_v[s]

    def start_gather(c, buf, sem):
        pltpu.async_copy(
            table_hbm.at[flatidx_v.at[pl.ds(c * L, L)]], buf, sem)

    def wait_gather(buf, sem):
        pltpu.make_async_copy(table_hbm.at[pl.ds(0, L)], buf, sem).wait()

    def start_scatter(c, buf, sem):
        pltpu.async_copy(buf, out_hbm.at[pl.ds(base + c * L, L)], sem)

    def wait_scatter(buf, sem):
        pltpu.make_async_copy(buf, out_hbm.at[pl.ds(base, L)], sem).wait()

    start_gather(0, rows0_v, gsem0)
    start_gather(1, rows1_v, gsem1)

    @pl.loop(0, CHUNKS - 2, step=2)
    def _main(c):
        wait_gather(rows0_v, gsem0)
        start_scatter(c, rows0_v, ssem0)
        wait_gather(rows1_v, gsem1)
        start_scatter(c + 1, rows1_v, ssem1)
        wait_scatter(rows0_v, ssem0)
        start_gather(c + 2, rows0_v, gsem0)
        wait_scatter(rows1_v, ssem1)
        start_gather(c + 3, rows1_v, gsem1)

    wait_gather(rows0_v, gsem0)
    start_scatter(CHUNKS - 2, rows0_v, ssem0)
    wait_gather(rows1_v, gsem1)
    start_scatter(CHUNKS - 1, rows1_v, ssem1)
    wait_scatter(rows0_v, ssem0)
    wait_scatter(rows1_v, ssem1)


def kernel(x_s, label, confounder_queue, sel_idx):
    bs = x_s.shape[0]
    table = confounder_queue.reshape(NUM_CLASSES * K, ROW)
    sel_flat = sel_idx.reshape(-1).astype(jnp.int32)
    label_rep = jnp.repeat(label.astype(jnp.int32), L)
    mesh = plsc.VectorSubcoreMesh(
        core_axis_name="c", subcore_axis_name="s",
        num_cores=NC, num_subcores=NS)
    run = pl.kernel(
        _gather_body,
        out_type=jax.ShapeDtypeStruct((bs * (NUM_CLASSES - 1) * N, ROW),
                                      jnp.float32),
        mesh=mesh,
        scratch_types=[
            pltpu.VMEM((BS * L,), jnp.int32),
            pltpu.VMEM((ROWS_PER_W,), jnp.int32),
            pltpu.VMEM((ROWS_PER_W,), jnp.int32),
            pltpu.VMEM((ROWS_PER_W,), jnp.int32),
            pltpu.VMEM((ROWS_PER_W,), jnp.int32),
            pltpu.VMEM((L, ROW), jnp.float32),
            pltpu.VMEM((L, ROW), jnp.float32),
            pltpu.SemaphoreType.DMA,
            pltpu.SemaphoreType.DMA,
            pltpu.SemaphoreType.DMA,
            pltpu.SemaphoreType.DMA,
        ],
    )
    out = run(table, label_rep, sel_flat)
    return out.reshape(bs, (NUM_CLASSES - 1) * N,
                       confounder_queue.shape[2], confounder_queue.shape[3],
                       confounder_queue.shape[4])


# speedup vs baseline: 5.7225x; 1.2600x over previous
import functools

import jax
import jax.numpy as jnp
from jax import lax
from jax.experimental import pallas as pl
from jax.experimental.pallas import tpu as pltpu
from jax.experimental.pallas import tpu_sc as plsc

NUM_CLASSES = 10
K = 2000
N = 16
NC, NS, L = 2, 16, 16
NW = NC * NS
BS = 64
M = (NUM_CLASSES - 1) * N
CH = 3 * 32
WDIM = 32
WQ = 4
WSPLIT = WDIM // WQ
POSITIONS = CH * WSPLIT
POS_PER_W = POSITIONS // NW


def _gather_body(bank_hbm, labelrep_hbm, sel_hbm, out_hbm,
                 labelrep_v, sel_v, slab_v, stage_v, osem):
    w = lax.axis_index("s") * NC + lax.axis_index("c")
    pltpu.sync_copy(labelrep_hbm, labelrep_v)
    pltpu.sync_copy(sel_hbm, sel_v)

    @pl.loop(0, POS_PER_W)
    def _pos(i):
        p = w * POS_PER_W + i
        ch = p // WSPLIT
        w0 = (p % WSPLIT) * WQ

        for cls in range(NUM_CLASSES):
            pltpu.sync_copy(
                bank_hbm.at[cls * CH + ch, pl.ds(w0, WQ), :], slab_v)

            @pl.loop(0, BS)
            def _sample(j):
                label_j = jnp.max(labelrep_v[pl.ds(j * L, L)])
                gt = (cls > label_j).astype(jnp.int32)
                moff = (cls - gt) * N
                kvec = sel_v[pl.ds(j * (NUM_CLASSES * N) + cls * N, N)]

                @pl.when(cls != label_j)
                def _():
                    for wr in range(WQ):
                        vals = plsc.load_gather(
                            slab_v, [jnp.full((L,), wr, jnp.int32), kvec])
                        stage_v[j, wr, pl.ds(moff, N)] = vals

        @pl.loop(0, BS)
        def _writeout(j):
            pltpu.async_copy(stage_v.at[j],
                             out_hbm.at[j, ch, pl.ds(w0, WQ), :], osem)

        @pl.loop(0, BS)
        def _drain(j):
            pltpu.make_async_copy(stage_v.at[j],
                                  out_hbm.at[j, ch, pl.ds(w0, WQ), :],
                                  osem).wait()


def kernel(x_s, label, confounder_queue, sel_idx):
    bs = x_s.shape[0]
    C, H, W = (confounder_queue.shape[2], confounder_queue.shape[3],
               confounder_queue.shape[4])
    bank = confounder_queue.transpose(0, 2, 3, 4, 1).reshape(
        NUM_CLASSES * C * H, W, K)
    sel_flat = sel_idx.reshape(-1).astype(jnp.int32)
    label_rep = jnp.repeat(label.astype(jnp.int32), L)
    mesh = plsc.VectorSubcoreMesh(
        core_axis_name="c", subcore_axis_name="s",
        num_cores=NC, num_subcores=NS)
    run = pl.kernel(
        _gather_body,
        out_type=jax.ShapeDtypeStruct((bs, C * H, W, M), jnp.float32),
        mesh=mesh,
        compiler_params=pltpu.CompilerParams(
            needs_layout_passes=False, use_tc_tiling_on_sc=True),
        scratch_types=[
            pltpu.VMEM((BS * L,), jnp.int32),
            pltpu.VMEM((BS * NUM_CLASSES * N,), jnp.int32),
            pltpu.VMEM((WQ, K), jnp.float32),
            pltpu.VMEM((BS, WQ, M), jnp.float32),
            pltpu.SemaphoreType.DMA,
        ],
    )
    out = run(bank, label_rep, sel_flat)
    return out.reshape(bs, C, H, W, M).transpose(0, 4, 1, 2, 3)


# speedup vs baseline: 11.2124x; 1.9594x over previous
import functools

import jax
import jax.numpy as jnp
from jax import lax
from jax.experimental import pallas as pl
from jax.experimental.pallas import tpu as pltpu
from jax.experimental.pallas import tpu_sc as plsc

NUM_CLASSES = 10
K = 2000
N = 16
NC, NS, L = 2, 16, 16
NW = NC * NS
BS = 64
M = (NUM_CLASSES - 1) * N
CH = 3 * 32
WDIM = 32
WQ = 4
WSPLIT = WDIM // WQ
POSITIONS = CH * WSPLIT
POS_PER_W = POSITIONS // NW
JC = BS * NUM_CLASSES


def _gather_body(bank_hbm, labelrep_hbm, sel_hbm, out_hbm,
                 labelrep_v, sel_v, midx_v, mask_v, slabA_v, slabB_v,
                 stage_v, semA, semB, osem):
    w = lax.axis_index("s") * NC + lax.axis_index("c")
    pltpu.sync_copy(labelrep_hbm, labelrep_v)
    pltpu.sync_copy(sel_hbm, sel_v)
    lanes = lax.iota(jnp.int32, L)

    @pl.loop(0, JC)
    def _tab(c2):
        j = c2 // NUM_CLASSES
        cls = c2 % NUM_CLASSES
        lbl = labelrep_v[pl.ds(j * L, L)]
        clsv = jnp.full((L,), cls, jnp.int32)
        gtv = (clsv > lbl).astype(jnp.int32)
        eqv = (clsv == lbl).astype(jnp.int32)
        midx_v[pl.ds(c2 * L, L)] = (clsv - gtv) * N + lanes
        mask_v[pl.ds(c2 * L, L)] = 1 - eqv

    slabs = (slabA_v, slabB_v)
    sems = (semA, semB)

    def slab_start(cls, ch, w0, b):
        pltpu.async_copy(bank_hbm.at[cls * CH + ch, pl.ds(w0, WQ), :],
                         slabs[b], sems[b])

    def slab_wait(ch, w0, b):
        pltpu.make_async_copy(bank_hbm.at[ch, pl.ds(w0, WQ), :],
                              slabs[b], sems[b]).wait()

    wfull = [jnp.full((L,), wr, jnp.int32) for wr in range(WQ)]

    @pl.loop(0, POS_PER_W)
    def _pos(i):
        p = w * POS_PER_W + i
        ch = p // WSPLIT
        w0 = (p % WSPLIT) * WQ

        slab_start(0, ch, w0, 0)
        for cls in range(NUM_CLASSES):
            b = cls % 2
            slab_wait(ch, w0, b)
            if cls + 1 < NUM_CLASSES:
                slab_start(cls + 1, ch, w0, 1 - b)

            @pl.loop(0, BS, unroll=4)
            def _sample(j):
                base = (j * NUM_CLASSES + cls) * L
                kvec = sel_v[pl.ds(base, L)]
                midx = midx_v[pl.ds(base, L)]
                mk = mask_v[pl.ds(base, L)] > 0
                jful = jnp.full((L,), j, jnp.int32)
                for wr in range(WQ):
                    vals = plsc.load_gather(slabs[b], [wfull[wr], kvec])
                    plsc.store_scatter(stage_v, [jful, wfull[wr], midx],
                                       vals, mask=mk)

        @pl.loop(0, BS)
        def _writeout(j):
            pltpu.async_copy(stage_v.at[j],
                             out_hbm.at[j, ch, pl.ds(w0, WQ), :], osem)

        @pl.loop(0, BS)
        def _drain(j):
            pltpu.make_async_copy(stage_v.at[j],
                                  out_hbm.at[j, ch, pl.ds(w0, WQ), :],
                                  osem).wait()


def kernel(x_s, label, confounder_queue, sel_idx):
    bs = x_s.shape[0]
    C, H, W = (confounder_queue.shape[2], confounder_queue.shape[3],
               confounder_queue.shape[4])
    bank = confounder_queue.transpose(0, 2, 3, 4, 1).reshape(
        NUM_CLASSES * C * H, W, K)
    sel_flat = sel_idx.reshape(-1).astype(jnp.int32)
    label_rep = jnp.repeat(label.astype(jnp.int32), L)
    mesh = plsc.VectorSubcoreMesh(
        core_axis_name="c", subcore_axis_name="s",
        num_cores=NC, num_subcores=NS)
    run = pl.kernel(
        _gather_body,
        out_type=jax.ShapeDtypeStruct((bs, C * H, W, M), jnp.float32),
        mesh=mesh,
        compiler_params=pltpu.CompilerParams(
            needs_layout_passes=False, use_tc_tiling_on_sc=True),
        scratch_types=[
            pltpu.VMEM((BS * L,), jnp.int32),
            pltpu.VMEM((JC * L,), jnp.int32),
            pltpu.VMEM((JC * L,), jnp.int32),
            pltpu.VMEM((JC * L,), jnp.int32),
            pltpu.VMEM((WQ, K), jnp.float32),
            pltpu.VMEM((WQ, K), jnp.float32),
            pltpu.VMEM((BS, WQ, M), jnp.float32),
            pltpu.SemaphoreType.DMA,
            pltpu.SemaphoreType.DMA,
            pltpu.SemaphoreType.DMA,
        ],
    )
    out = run(bank, label_rep, sel_flat)
    return out.reshape(bs, C, H, W, M).transpose(0, 4, 1, 2, 3)
